# baseline (device time: 51847 ns/iter reference)
import jax
import jax.numpy as jnp
from jax import lax
from jax.experimental import pallas as pl
from jax.experimental.pallas import tpu as pltpu

N_RING = 8
N_Q = 4
CW_DEPTH = (4, 4, 2, 2)
CCW_DEPTH = (2, 2, 4, 4)
N_XF = 4
XF_SEND = {
    ("cw", 0, 2): 0,
    ("cw", 1, 2): 1,
    ("ccw", 2, 2): 2,
    ("ccw", 3, 2): 3,
}
XF_RECV = [
    (2, -3), (3, -3), (0, +3), (1, +3),
]


def kernel(x):
    m, n = x.shape
    rows = m // N_RING
    qrows = rows // N_Q

    def body(x_hbm, out_hbm, gbuf, xchunk, mysend, p1recv, copy_sem,
             out_sems, *sems):
        p1_send = sems[0]
        p1_recv = sems[1]
        cw_send = sems[2:2 + N_Q]
        cw_recv = sems[2 + N_Q:2 + 2 * N_Q]
        ccw_send = sems[2 + 2 * N_Q:2 + 3 * N_Q]
        ccw_recv = sems[2 + 3 * N_Q:2 + 4 * N_Q]
        xf_send = sems[2 + 4 * N_Q]
        xf_recv = sems[3 + 4 * N_Q]

        my_x = lax.axis_index("x")
        my_y = lax.axis_index("y")
        my_z = lax.axis_index("z")
        partner = (1 - my_x, my_y, my_z)

        def pq(l):
            return (l + 2 * my_x) % N_Q

        r = jnp.where(my_y == 0, my_z, 7 - my_z)

        def ring_coords(p):
            p = p % N_RING
            py = (p >= 4).astype(my_z.dtype)
            pz = jnp.where(p < 4, p, 7 - p)
            return (my_x, py, pz)

        nxt = ring_coords(r + 1)
        prv = ring_coords(r - 1)

        def qs(k, q):
            return pl.ds((k % N_RING) * rows + q * qrows, qrows)

        def cs(k):
            return pl.ds((k % N_RING) * rows, rows)

        cp = pltpu.make_async_copy(x_hbm.at[cs(r)], xchunk, copy_sem)
        cp.start()

        barrier_sem = pltpu.get_barrier_semaphore()
        for dev in (partner, nxt, prv):
            pl.semaphore_signal(
                barrier_sem, inc=1, device_id=dev,
                device_id_type=pl.DeviceIdType.MESH,
            )
        pl.semaphore_wait(barrier_sem, 3)

        cp.wait()
        mysend[...] = xchunk[...].astype(jnp.bfloat16)

        sends = []
        out_copies = []

        def start(d):
            d.start()
            sends.append(d)

        def flush_chunk(k):
            d = pltpu.make_async_copy(
                gbuf.at[cs(k)], out_hbm.at[cs(k)],
                out_sems.at[len(out_copies)],
            )
            d.start()
            out_copies.append(d)

        def rcopy(slc, send_sems, recv_sems, idx, dev):
            return pltpu.make_async_remote_copy(
                src_ref=gbuf.at[slc],
                dst_ref=gbuf.at[slc],
                send_sem=send_sems.at[idx],
                recv_sem=recv_sems.at[idx],
                device_id=dev,
                device_id_type=pl.DeviceIdType.MESH,
            )

        p1 = []
        for i in range(N_Q):
            send_q = (i + 2 * (1 - my_x)) % N_Q
            sl = pl.ds(send_q * qrows, qrows)
            d = pltpu.make_async_remote_copy(
                src_ref=mysend.at[sl],
                dst_ref=p1recv.at[sl],
                send_sem=p1_send.at[i],
                recv_sem=p1_recv.at[i],
                device_id=partner,
                device_id_type=pl.DeviceIdType.MESH,
            )
            start(d)
            p1.append(d)

        for l in range(N_Q):
            p1[l].wait_recv()
            q = pq(l)
            sl = pl.ds(q * qrows, qrows)
            gbuf[qs(r, q), :] = (
                xchunk[sl, :] + p1recv[sl, :].astype(jnp.float32)
            ).astype(jnp.bfloat16)
            start(rcopy(qs(r, q), cw_send[l], cw_recv[l], 0, nxt))
            start(rcopy(qs(r, q), ccw_send[l], ccw_recv[l], 0, prv))
        flush_chunk(r)

        lanes = [
            ("cw", 0), ("cw", 1), ("ccw", 2), ("ccw", 3),
            ("cw", 2), ("cw", 3), ("ccw", 0), ("ccw", 1),
        ]
        for j in range(4):
            for dirn, l in lanes:
                if dirn == "cw":
                    depth, dev = CW_DEPTH[l], nxt
                    ssem, rsem = cw_send[l], cw_recv[l]
                    k = r - 1 - j
                else:
                    depth, dev = CCW_DEPTH[l], prv
                    ssem, rsem = ccw_send[l], ccw_recv[l]
                    k = r + 1 + j
                if j < depth:
                    rcopy(qs(k, pq(l)), ssem, rsem, j, dev).wait_recv()
                    if j + 1 < depth:
                        start(rcopy(qs(k, pq(l)), ssem, rsem, j + 1, dev))
                    xi = XF_SEND.get((dirn, l, j))
                    if xi is not None:
                        start(rcopy(qs(k, pq(l)), xf_send, xf_recv, xi, partner))
            if j == 0:
                flush_chunk(r - 1)
                flush_chunk(r + 1)
            elif j == 1:
                flush_chunk(r - 2)
                flush_chunk(r + 2)
            elif j == 3:
                flush_chunk(r - 4)

        for i, (l, off) in enumerate(XF_RECV):
            rcopy(qs(r + off, pq(l)), xf_send, xf_recv, i, partner).wait_recv()
            if i == 1:
                flush_chunk(r - 3)
            elif i == 3:
                flush_chunk(r + 3)

        for d in sends:
            d.wait_send()
        for d in out_copies:
            d.wait()

    qsem = pltpu.SemaphoreType.DMA
    return pl.pallas_call(
        body,
        out_shape=jax.ShapeDtypeStruct((m, n), jnp.bfloat16),
        in_specs=[pl.BlockSpec(memory_space=pl.ANY)],
        out_specs=pl.BlockSpec(memory_space=pl.ANY),
        scratch_shapes=[
            pltpu.VMEM((m, n), jnp.bfloat16),
            pltpu.VMEM((rows, n), jnp.float32),
            pltpu.VMEM((rows, n), jnp.bfloat16),
            pltpu.VMEM((rows, n), jnp.bfloat16),
            qsem,
            qsem((N_RING,)),
            qsem((N_Q,)), qsem((N_Q,)),
            *[qsem((CW_DEPTH[l],)) for l in range(N_Q)],
            *[qsem((CW_DEPTH[l],)) for l in range(N_Q)],
            *[qsem((CCW_DEPTH[l],)) for l in range(N_Q)],
            *[qsem((CCW_DEPTH[l],)) for l in range(N_Q)],
            qsem((N_XF,)), qsem((N_XF,)),
        ],
        compiler_params=pltpu.CompilerParams(collective_id=0),
    )(x)


# device time: 51662 ns/iter; 1.0036x vs baseline; 1.0036x over previous
import jax
import jax.numpy as jnp
from jax import lax
from jax.experimental import pallas as pl
from jax.experimental.pallas import tpu as pltpu

N_RING = 8
N_Q = 4
CW_DEPTH = (4, 4, 2, 2)
CCW_DEPTH = (2, 2, 4, 4)
N_XF = 4
XF_SEND = {
    ("cw", 0, 2): 0,
    ("cw", 1, 2): 1,
    ("ccw", 2, 2): 2,
    ("ccw", 3, 2): 3,
}
XF_RECV = [
    (2, -3), (3, -3), (0, +3), (1, +3),
]


def kernel(x):
    m, n = x.shape
    rows = m // N_RING
    qrows = rows // N_Q

    def body(x_hbm, out_ref, xchunk, mysend, p1recv, copy_sem, *sems):
        p1_send = sems[0]
        p1_recv = sems[1]
        cw_send = sems[2:2 + N_Q]
        cw_recv = sems[2 + N_Q:2 + 2 * N_Q]
        ccw_send = sems[2 + 2 * N_Q:2 + 3 * N_Q]
        ccw_recv = sems[2 + 3 * N_Q:2 + 4 * N_Q]
        xf_send = sems[2 + 4 * N_Q]
        xf_recv = sems[3 + 4 * N_Q]

        my_x = lax.axis_index("x")
        my_y = lax.axis_index("y")
        my_z = lax.axis_index("z")
        partner = (1 - my_x, my_y, my_z)

        def pq(l):
            return (l + 2 * my_x) % N_Q

        r = jnp.where(my_y == 0, my_z, 7 - my_z)

        def ring_coords(p):
            p = p % N_RING
            py = (p >= 4).astype(my_z.dtype)
            pz = jnp.where(p < 4, p, 7 - p)
            return (my_x, py, pz)

        nxt = ring_coords(r + 1)
        prv = ring_coords(r - 1)

        def qs(k, q):
            return pl.ds((k % N_RING) * rows + q * qrows, qrows)

        my_rows = pl.ds(r * rows, rows)
        cp = pltpu.make_async_copy(x_hbm.at[my_rows], xchunk, copy_sem)
        cp.start()

        barrier_sem = pltpu.get_barrier_semaphore()
        for dev in (partner, nxt, prv):
            pl.semaphore_signal(
                barrier_sem, inc=1, device_id=dev,
                device_id_type=pl.DeviceIdType.MESH,
            )
        pl.semaphore_wait(barrier_sem, 3)

        cp.wait()
        mysend[...] = xchunk[...].astype(jnp.bfloat16)

        sends = []

        def start(d):
            d.start()
            sends.append(d)

        def rcopy(slc, send_sems, recv_sems, idx, dev):
            return pltpu.make_async_remote_copy(
                src_ref=out_ref.at[slc],
                dst_ref=out_ref.at[slc],
                send_sem=send_sems.at[idx],
                recv_sem=recv_sems.at[idx],
                device_id=dev,
                device_id_type=pl.DeviceIdType.MESH,
            )

        p1 = []
        for i in range(N_Q):
            send_q = (i + 2 * (1 - my_x)) % N_Q
            sl = pl.ds(send_q * qrows, qrows)
            d = pltpu.make_async_remote_copy(
                src_ref=mysend.at[sl],
                dst_ref=p1recv.at[sl],
                send_sem=p1_send.at[i],
                recv_sem=p1_recv.at[i],
                device_id=partner,
                device_id_type=pl.DeviceIdType.MESH,
            )
            start(d)
            p1.append(d)

        for l in range(N_Q):
            p1[l].wait_recv()
            q = pq(l)
            sl = pl.ds(q * qrows, qrows)
            out_ref[qs(r, q), :] = (
                xchunk[sl, :] + p1recv[sl, :].astype(jnp.float32)
            ).astype(jnp.bfloat16)
            start(rcopy(qs(r, q), cw_send[l], cw_recv[l], 0, nxt))
            start(rcopy(qs(r, q), ccw_send[l], ccw_recv[l], 0, prv))

        lanes = [
            ("cw", 0), ("cw", 1), ("ccw", 2), ("ccw", 3),
            ("cw", 2), ("cw", 3), ("ccw", 0), ("ccw", 1),
        ]
        for j in range(4):
            for dirn, l in lanes:
                if dirn == "cw":
                    depth, dev = CW_DEPTH[l], nxt
                    ssem, rsem = cw_send[l], cw_recv[l]
                    k = r - 1 - j
                else:
                    depth, dev = CCW_DEPTH[l], prv
                    ssem, rsem = ccw_send[l], ccw_recv[l]
                    k = r + 1 + j
                if j < depth:
                    rcopy(qs(k, pq(l)), ssem, rsem, j, dev).wait_recv()
                    if j + 1 < depth:
                        start(rcopy(qs(k, pq(l)), ssem, rsem, j + 1, dev))
                    xi = XF_SEND.get((dirn, l, j))
                    if xi is not None:
                        start(rcopy(qs(k, pq(l)), xf_send, xf_recv, xi, partner))

        for i, (l, off) in enumerate(XF_RECV):
            rcopy(qs(r + off, pq(l)), xf_send, xf_recv, i, partner).wait_recv()

        for d in sends:
            d.wait_send()

    qsem = pltpu.SemaphoreType.DMA
    return pl.pallas_call(
        body,
        out_shape=jax.ShapeDtypeStruct((m, n), jnp.bfloat16),
        in_specs=[pl.BlockSpec(memory_space=pl.ANY)],
        out_specs=pl.BlockSpec(memory_space=pltpu.VMEM),
        scratch_shapes=[
            pltpu.VMEM((rows, n), jnp.float32),
            pltpu.VMEM((rows, n), jnp.bfloat16),
            pltpu.VMEM((rows, n), jnp.bfloat16),
            qsem,
            qsem((N_Q,)), qsem((N_Q,)),
            *[qsem((CW_DEPTH[l],)) for l in range(N_Q)],
            *[qsem((CW_DEPTH[l],)) for l in range(N_Q)],
            *[qsem((CCW_DEPTH[l],)) for l in range(N_Q)],
            *[qsem((CCW_DEPTH[l],)) for l in range(N_Q)],
            qsem((N_XF,)), qsem((N_XF,)),
        ],
        compiler_params=pltpu.CompilerParams(collective_id=0),
    )(x)
